# baseline (speedup 1.0000x reference)
import functools

import jax
import jax.numpy as jnp
from jax import lax
from jax.experimental import pallas as pl
from jax.experimental.pallas import tpu as pltpu
from jax.experimental.pallas import tpu_sc as plsc

B = 16384
L = 20
EMB = 32
H1 = 64
H2 = 32

NC = 2
NS = 16
NW = NC * NS
LANES = 16
PADW = 128

ITEMS_PER_W = B // NW
CHUNK_ITEMS = 64
N_CHUNKS = ITEMS_PER_W // CHUNK_ITEMS
TID_ROWS_W = ITEMS_PER_W // 128

_MESH = plsc.VectorSubcoreMesh(core_axis_name="c", subcore_axis_name="s")
_SC_PARAMS = pltpu.CompilerParams(use_tc_tiling_on_sc=False)


def _sc_token_pool(tokT, text_table):
    @functools.partial(
        pl.kernel,
        mesh=_MESH,
        compiler_params=_SC_PARAMS,
        out_type=jax.ShapeDtypeStruct((B, PADW), jnp.float32),
        scratch_types=[
            pltpu.VMEM((2, L, CHUNK_ITEMS), jnp.int32),
            pltpu.VMEM((2, L * CHUNK_ITEMS, EMB), jnp.float32),
            pltpu.VMEM((N_CHUNKS, CHUNK_ITEMS), jnp.int32),
            pltpu.VMEM((CHUNK_ITEMS, EMB), jnp.float32),
            pltpu.VMEM_SHARED((NS * ITEMS_PER_W, EMB), jnp.float32),
            pltpu.SemaphoreType.DMA,
            pltpu.SemaphoreType.DMA,
            pltpu.SemaphoreType.DMA,
            pltpu.SemaphoreType.DMA,
        ],
    )
    def k(tokT_hbm, xtab_hbm, sums_out, kidx_v, krows_v, dest_v, zeros_v,
          acc_sh, semg0, semg1, sems, semo):
        cid = lax.axis_index("c")
        sid = lax.axis_index("s")
        wid = cid * NS + sid
        semg = (semg0, semg1)

        iota = lax.iota(jnp.int32, LANES)
        zvec = jnp.zeros((LANES,), jnp.float32)

        for c in range(N_CHUNKS):
            for q in range(CHUNK_ITEMS // LANES):
                dest_v[c, pl.ds(q * LANES, LANES)] = (
                    iota + (q * LANES + c * CHUNK_ITEMS) + sid * ITEMS_PER_W
                )

        for a in range(CHUNK_ITEMS):
            for b in range(EMB // LANES):
                zeros_v[a, pl.ds(b * LANES, LANES)] = zvec
        for c in range(N_CHUNKS):
            pltpu.sync_copy(
                zeros_v,
                acc_sh.at[pl.ds(sid * ITEMS_PER_W + c * CHUNK_ITEMS, CHUNK_ITEMS)],
            )

        def load_idx(c, slot):
            pltpu.sync_copy(
                tokT_hbm.at[:, pl.ds(wid * ITEMS_PER_W + c * CHUNK_ITEMS,
                                     CHUNK_ITEMS)],
                kidx_v.at[slot],
            )

        def fire_gathers(slot):
            return [
                pltpu.async_copy(
                    xtab_hbm.at[kidx_v.at[slot, l]],
                    krows_v.at[slot, pl.ds(l * CHUNK_ITEMS, CHUNK_ITEMS)],
                    semg[slot],
                )
                for l in range(L)
            ]

        load_idx(0, 0)
        h0 = fire_gathers(0)
        load_idx(1, 1)
        h1 = fire_gathers(1)
        handles = [h0, h1]

        @pl.loop(0, N_CHUNKS // 2)
        def _(i):
            for b in range(2):
                c = i * 2 + b
                for cp in handles[b]:
                    cp.wait()
                scs = [
                    pltpu.async_copy(
                        krows_v.at[b, pl.ds(l * CHUNK_ITEMS, CHUNK_ITEMS)],
                        acc_sh.at[dest_v.at[c]],
                        sems,
                        add=True,
                    )
                    for l in range(L)
                ]
                for cp in scs:
                    cp.wait()
                @pl.when(c + 2 < N_CHUNKS)
                def _():
                    load_idx(c + 2, b)
                    fire_gathers(b)

        pltpu.async_copy(
            acc_sh.at[pl.ds(sid * ITEMS_PER_W, ITEMS_PER_W)],
            sums_out.at[pl.ds(wid * ITEMS_PER_W, ITEMS_PER_W), pl.ds(0, EMB)],
            semo,
        ).wait()

    return k(tokT, text_table)


def _sc_title_gather(tid3, title_table_bf):
    @functools.partial(
        pl.kernel,
        mesh=_MESH,
        compiler_params=_SC_PARAMS,
        out_type=jax.ShapeDtypeStruct((B, EMB), jnp.bfloat16),
        scratch_types=[
            pltpu.VMEM((TID_ROWS_W, 128), jnp.int32),
            pltpu.VMEM((ITEMS_PER_W, EMB), jnp.bfloat16),
            pltpu.SemaphoreType.DMA,
        ],
    )
    def k(tid_hbm, ttab_hbm, title_out, tidx_v, trows_v, sem):
        cid = lax.axis_index("c")
        sid = lax.axis_index("s")
        wid = cid * NS + sid

        pltpu.sync_copy(tid_hbm.at[wid], tidx_v)
        tcps = [
            pltpu.async_copy(
                ttab_hbm.at[tidx_v.at[j]],
                trows_v.at[pl.ds(j * 128, 128)],
                sem,
            )
            for j in range(TID_ROWS_W)
        ]
        for cp in tcps:
            cp.wait()
        pltpu.sync_copy(
            trows_v,
            title_out.at[pl.ds(wid * ITEMS_PER_W, ITEMS_PER_W)],
        )

    return k(tid3, title_table_bf)


BLK = 2048


def _mlp_body(title_ref, sums_ref, tokT_ref, xtab_ref,
              w1a_ref, w1b_ref, b1_ref, w2_ref, b2_ref, out_ref):
    mask = (tokT_ref[...] != 0).astype(jnp.float32)
    ones = jnp.ones((L, 1), jnp.float32)
    cnt = jax.lax.dot_general(mask, ones, (((0,), (0,)), ((), ())),
                              preferred_element_type=jnp.float32)
    row0 = xtab_ref[0:1, :]
    title = title_ref[...]
    sums = sums_ref[:, :EMB]
    text = (sums - (L - cnt) * row0) / jnp.maximum(cnt, 1.0)
    bf = jnp.bfloat16
    h = jnp.dot(title, w1a_ref[...].astype(bf),
                preferred_element_type=jnp.float32)
    h += jnp.dot(text.astype(bf), w1b_ref[...].astype(bf),
                 preferred_element_type=jnp.float32)
    h = jnp.maximum(h + b1_ref[...], 0.0)
    out_ref[...] = (
        jnp.dot(h.astype(bf), w2_ref[...].astype(bf),
                preferred_element_type=jnp.float32) + b2_ref[...]
    )


def _tc_mlp(title_emb, sums, tokT, text_table, w1a, w1b, b1, w2, b2):
    grid = B // BLK
    return pl.pallas_call(
        _mlp_body,
        grid=(grid,),
        in_specs=[
            pl.BlockSpec((BLK, EMB), lambda i: (i, 0)),
            pl.BlockSpec((BLK, PADW), lambda i: (i, 0)),
            pl.BlockSpec((L, BLK), lambda i: (0, i)),
            pl.BlockSpec((8, EMB), lambda i: (0, 0)),
            pl.BlockSpec((EMB, H1), lambda i: (0, 0)),
            pl.BlockSpec((EMB, H1), lambda i: (0, 0)),
            pl.BlockSpec((1, H1), lambda i: (0, 0)),
            pl.BlockSpec((H1, H2), lambda i: (0, 0)),
            pl.BlockSpec((1, H2), lambda i: (0, 0)),
        ],
        out_specs=pl.BlockSpec((BLK, H2), lambda i: (i, 0)),
        out_shape=jax.ShapeDtypeStruct((B, H2), jnp.float32),
    )(title_emb, sums, tokT, text_table, w1a, w1b, b1, w2, b2)


def kernel(title_ids, token_ids, title_table, text_table, W1, b1, W2, b2):
    tid3 = title_ids.astype(jnp.int32).reshape(NW, TID_ROWS_W, 128)
    tokT = token_ids.astype(jnp.int32).T
    sums = _sc_token_pool(tokT, text_table)
    title_emb = _sc_title_gather(tid3, title_table.astype(jnp.bfloat16))
    return _tc_mlp(
        title_emb, sums, tokT, text_table,
        W1[:EMB], W1[EMB:], b1.reshape(1, H1), W2, b2.reshape(1, H2),
    )


# speedup vs baseline: 1.4297x; 1.4297x over previous
import functools

import jax
import jax.numpy as jnp
from jax import lax
from jax.experimental import pallas as pl
from jax.experimental.pallas import tpu as pltpu
from jax.experimental.pallas import tpu_sc as plsc

B = 16384
L = 20
EMB = 32
H1 = 64
H2 = 32

NC = 2
NS = 16
NW = NC * NS
LANES = 16
PADW = 128

ITEMS_PER_W = B // NW
CHUNK_ITEMS = 64
N_CHUNKS = ITEMS_PER_W // CHUNK_ITEMS
TID_ROWS_W = ITEMS_PER_W // 128

_MESH = plsc.VectorSubcoreMesh(core_axis_name="c", subcore_axis_name="s")
_SC_PARAMS = pltpu.CompilerParams(use_tc_tiling_on_sc=False)


def _sc_token_pool(tokT, text_table):
    @functools.partial(
        pl.kernel,
        mesh=_MESH,
        compiler_params=_SC_PARAMS,
        out_type=jax.ShapeDtypeStruct((B, PADW), jnp.float32),
        scratch_types=[
            pltpu.VMEM((2, L, CHUNK_ITEMS), jnp.int32),
            pltpu.VMEM((2, L * CHUNK_ITEMS, EMB), jnp.float32),
            pltpu.VMEM((N_CHUNKS, CHUNK_ITEMS), jnp.int32),
            pltpu.VMEM((CHUNK_ITEMS, EMB), jnp.float32),
            pltpu.VMEM_SHARED((NS * ITEMS_PER_W, EMB), jnp.float32),
            pltpu.SemaphoreType.DMA,
            pltpu.SemaphoreType.DMA,
            pltpu.SemaphoreType.DMA,
            pltpu.SemaphoreType.DMA,
        ],
    )
    def k(tokT_hbm, xtab_hbm, sums_out, kidx_v, krows_v, dest_v, zeros_v,
          acc_sh, semg0, semg1, sems, semo):
        cid = lax.axis_index("c")
        sid = lax.axis_index("s")
        wid = cid * NS + sid
        semg = (semg0, semg1)

        iota = lax.iota(jnp.int32, LANES)
        zvec = jnp.zeros((LANES,), jnp.float32)

        for c in range(N_CHUNKS):
            for q in range(CHUNK_ITEMS // LANES):
                dest_v[c, pl.ds(q * LANES, LANES)] = (
                    iota + (q * LANES + c * CHUNK_ITEMS) + sid * ITEMS_PER_W
                )

        for a in range(CHUNK_ITEMS):
            for b in range(EMB // LANES):
                zeros_v[a, pl.ds(b * LANES, LANES)] = zvec
        for c in range(N_CHUNKS):
            pltpu.sync_copy(
                zeros_v,
                acc_sh.at[pl.ds(sid * ITEMS_PER_W + c * CHUNK_ITEMS, CHUNK_ITEMS)],
            )

        def load_idx(c, slot):
            pltpu.sync_copy(
                tokT_hbm.at[:, pl.ds(wid * ITEMS_PER_W + c * CHUNK_ITEMS,
                                     CHUNK_ITEMS)],
                kidx_v.at[slot],
            )

        def fire_gathers(slot):
            return [
                pltpu.async_copy(
                    xtab_hbm.at[kidx_v.at[slot, l]],
                    krows_v.at[slot, pl.ds(l * CHUNK_ITEMS, CHUNK_ITEMS)],
                    semg[slot],
                )
                for l in range(L)
            ]

        load_idx(0, 0)
        h0 = fire_gathers(0)
        load_idx(1, 1)
        h1 = fire_gathers(1)
        handles = [h0, h1]

        @pl.loop(0, N_CHUNKS // 2)
        def _(i):
            for b in range(2):
                c = i * 2 + b
                for cp in handles[b]:
                    cp.wait()
                scs = [
                    pltpu.async_copy(
                        krows_v.at[b, pl.ds(l * CHUNK_ITEMS, CHUNK_ITEMS)],
                        acc_sh.at[dest_v.at[c]],
                        sems,
                        add=True,
                    )
                    for l in range(L)
                ]
                for cp in scs:
                    cp.wait()
                @pl.when(c + 2 < N_CHUNKS)
                def _():
                    load_idx(c + 2, b)
                    fire_gathers(b)

        pltpu.async_copy(
            acc_sh.at[pl.ds(sid * ITEMS_PER_W, ITEMS_PER_W)],
            sums_out.at[pl.ds(wid * ITEMS_PER_W, ITEMS_PER_W), pl.ds(0, EMB)],
            semo,
        ).wait()

    return k(tokT, text_table)


def _sc_title_gather(tid3, title_table):
    @functools.partial(
        pl.kernel,
        mesh=_MESH,
        compiler_params=_SC_PARAMS,
        out_type=jax.ShapeDtypeStruct((B, PADW), jnp.float32),
        scratch_types=[
            pltpu.VMEM((TID_ROWS_W, 128), jnp.int32),
            pltpu.VMEM((ITEMS_PER_W, EMB), jnp.float32),
            pltpu.SemaphoreType.DMA,
        ],
    )
    def k(tid_hbm, ttab_hbm, title_out, tidx_v, trows_v, sem):
        cid = lax.axis_index("c")
        sid = lax.axis_index("s")
        wid = cid * NS + sid

        pltpu.sync_copy(tid_hbm.at[wid], tidx_v)
        tcps = [
            pltpu.async_copy(
                ttab_hbm.at[tidx_v.at[j]],
                trows_v.at[pl.ds(j * 128, 128)],
                sem,
            )
            for j in range(TID_ROWS_W)
        ]
        for cp in tcps:
            cp.wait()
        pltpu.sync_copy(
            trows_v,
            title_out.at[pl.ds(wid * ITEMS_PER_W, ITEMS_PER_W), pl.ds(0, EMB)],
        )

    return k(tid3, title_table)


BLK = 1024


def _mlp_body(title_ref, sums_ref, tokT_ref, xtab_ref,
              w1a_ref, w1b_ref, b1_ref, w2_ref, b2_ref, out_ref):
    mask = (tokT_ref[...] != 0).astype(jnp.float32)
    ones = jnp.ones((L, 1), jnp.float32)
    cnt = jax.lax.dot_general(mask, ones, (((0,), (0,)), ((), ())),
                              preferred_element_type=jnp.float32)
    row0 = xtab_ref[0:1, :]
    title = title_ref[:, :EMB]
    sums = sums_ref[:, :EMB]
    text = (sums - (L - cnt) * row0) / jnp.maximum(cnt, 1.0)
    bf = jnp.bfloat16
    h = jnp.dot(title.astype(bf), w1a_ref[...].astype(bf),
                preferred_element_type=jnp.float32)
    h += jnp.dot(text.astype(bf), w1b_ref[...].astype(bf),
                 preferred_element_type=jnp.float32)
    h = jnp.maximum(h + b1_ref[...], 0.0)
    out_ref[...] = (
        jnp.dot(h.astype(bf), w2_ref[...].astype(bf),
                preferred_element_type=jnp.float32) + b2_ref[...]
    )


def _tc_mlp(title_emb, sums, tokT, text_table, w1a, w1b, b1, w2, b2):
    grid = B // BLK
    return pl.pallas_call(
        _mlp_body,
        grid=(grid,),
        in_specs=[
            pl.BlockSpec((BLK, PADW), lambda i: (i, 0)),
            pl.BlockSpec((BLK, PADW), lambda i: (i, 0)),
            pl.BlockSpec((L, BLK), lambda i: (0, i)),
            pl.BlockSpec((8, EMB), lambda i: (0, 0)),
            pl.BlockSpec((EMB, H1), lambda i: (0, 0)),
            pl.BlockSpec((EMB, H1), lambda i: (0, 0)),
            pl.BlockSpec((1, H1), lambda i: (0, 0)),
            pl.BlockSpec((H1, H2), lambda i: (0, 0)),
            pl.BlockSpec((1, H2), lambda i: (0, 0)),
        ],
        out_specs=pl.BlockSpec((BLK, H2), lambda i: (i, 0)),
        out_shape=jax.ShapeDtypeStruct((B, H2), jnp.float32),
    )(title_emb, sums, tokT, text_table, w1a, w1b, b1, w2, b2)


def kernel(title_ids, token_ids, title_table, text_table, W1, b1, W2, b2):
    tid3 = title_ids.astype(jnp.int32).reshape(NW, TID_ROWS_W, 128)
    tokT = token_ids.astype(jnp.int32).T
    sums = _sc_token_pool(tokT, text_table)
    title_emb = _sc_title_gather(tid3, title_table)
    return _tc_mlp(
        title_emb, sums, tokT, text_table,
        W1[:EMB], W1[EMB:], b1.reshape(1, H1), W2, b2.reshape(1, H2),
    )


# speedup vs baseline: 1.5316x; 1.0712x over previous
import functools

import jax
import jax.numpy as jnp
from jax import lax
from jax.experimental import pallas as pl
from jax.experimental.pallas import tpu as pltpu
from jax.experimental.pallas import tpu_sc as plsc

B = 16384
L = 20
EMB = 32
H1 = 64
H2 = 32

NC = 2
NS = 16
NW = NC * NS
LANES = 16
PADW = 128

ITEMS_PER_W = B // NW
CHUNK_ITEMS = 64
N_CHUNKS = ITEMS_PER_W // CHUNK_ITEMS
TID_ROWS_W = ITEMS_PER_W // 128

_MESH = plsc.VectorSubcoreMesh(core_axis_name="c", subcore_axis_name="s")
_SC_PARAMS = pltpu.CompilerParams(use_tc_tiling_on_sc=False)


def _sc_token_pool(tokT, text_table):
    @functools.partial(
        pl.kernel,
        mesh=_MESH,
        compiler_params=_SC_PARAMS,
        out_type=jax.ShapeDtypeStruct((B, PADW), jnp.float32),
        scratch_types=[
            pltpu.VMEM((2, L, CHUNK_ITEMS), jnp.int32),
            pltpu.VMEM((2, L * CHUNK_ITEMS, EMB), jnp.float32),
            pltpu.VMEM((N_CHUNKS, CHUNK_ITEMS), jnp.int32),
            pltpu.VMEM((CHUNK_ITEMS, EMB), jnp.float32),
            pltpu.VMEM_SHARED((NS * ITEMS_PER_W, EMB), jnp.float32),
            pltpu.SemaphoreType.DMA,
            pltpu.SemaphoreType.DMA,
            pltpu.SemaphoreType.DMA,
            pltpu.SemaphoreType.DMA,
        ],
    )
    def k(tokT_hbm, xtab_hbm, sums_out, kidx_v, krows_v, dest_v, zeros_v,
          acc_sh, semg0, semg1, sems, semo):
        cid = lax.axis_index("c")
        sid = lax.axis_index("s")
        wid = cid * NS + sid
        semg = (semg0, semg1)

        iota = lax.iota(jnp.int32, LANES)
        zvec = jnp.zeros((LANES,), jnp.float32)

        for c in range(N_CHUNKS):
            for q in range(CHUNK_ITEMS // LANES):
                dest_v[c, pl.ds(q * LANES, LANES)] = (
                    iota + (q * LANES + c * CHUNK_ITEMS) + sid * ITEMS_PER_W
                )

        for a in range(CHUNK_ITEMS):
            for b in range(EMB // LANES):
                zeros_v[a, pl.ds(b * LANES, LANES)] = zvec
        for c in range(N_CHUNKS):
            pltpu.sync_copy(
                zeros_v,
                acc_sh.at[pl.ds(sid * ITEMS_PER_W + c * CHUNK_ITEMS, CHUNK_ITEMS)],
            )

        def load_idx(c, slot):
            pltpu.sync_copy(
                tokT_hbm.at[:, pl.ds(wid * ITEMS_PER_W + c * CHUNK_ITEMS,
                                     CHUNK_ITEMS)],
                kidx_v.at[slot],
            )

        def fire_gathers(slot):
            return [
                pltpu.async_copy(
                    xtab_hbm.at[kidx_v.at[slot, l]],
                    krows_v.at[slot, pl.ds(l * CHUNK_ITEMS, CHUNK_ITEMS)],
                    semg[slot],
                )
                for l in range(L)
            ]

        load_idx(0, 0)
        h0 = fire_gathers(0)
        load_idx(1, 1)
        h1 = fire_gathers(1)
        handles = [h0, h1]

        @pl.loop(0, N_CHUNKS // 2)
        def _(i):
            for b in range(2):
                c = i * 2 + b
                for cp in handles[b]:
                    cp.wait()
                scs = [
                    pltpu.async_copy(
                        krows_v.at[b, pl.ds(l * CHUNK_ITEMS, CHUNK_ITEMS)],
                        acc_sh.at[dest_v.at[c]],
                        sems,
                        add=True,
                    )
                    for l in range(L)
                ]
                for cp in scs:
                    cp.wait()
                @pl.when(c + 2 < N_CHUNKS)
                def _():
                    load_idx(c + 2, b)
                    fire_gathers(b)

        pltpu.async_copy(
            acc_sh.at[pl.ds(sid * ITEMS_PER_W, ITEMS_PER_W)],
            sums_out.at[pl.ds(wid * ITEMS_PER_W, ITEMS_PER_W), pl.ds(0, EMB)],
            semo,
        ).wait()

    return k(tokT, text_table)


def _sc_title_gather(tid3, title_table):
    @functools.partial(
        pl.kernel,
        mesh=_MESH,
        compiler_params=_SC_PARAMS,
        out_type=jax.ShapeDtypeStruct((B, PADW), jnp.float32),
        scratch_types=[
            pltpu.VMEM((TID_ROWS_W, 128), jnp.int32),
            pltpu.VMEM((ITEMS_PER_W, EMB), jnp.float32),
            pltpu.SemaphoreType.DMA,
        ],
    )
    def k(tid_hbm, ttab_hbm, title_out, tidx_v, trows_v, sem):
        cid = lax.axis_index("c")
        sid = lax.axis_index("s")
        wid = cid * NS + sid

        pltpu.sync_copy(tid_hbm.at[wid], tidx_v)
        tcps = [
            pltpu.async_copy(
                ttab_hbm.at[tidx_v.at[j]],
                trows_v.at[pl.ds(j * 128, 128)],
                sem,
            )
            for j in range(TID_ROWS_W)
        ]
        for cp in tcps:
            cp.wait()
        pltpu.sync_copy(
            trows_v,
            title_out.at[pl.ds(wid * ITEMS_PER_W, ITEMS_PER_W), pl.ds(0, EMB)],
        )

    return k(tid3, title_table)


BLK = 4096


def _mlp_body(title_ref, sums_ref, tokT_ref, xtab_ref,
              w1a_ref, w1b_ref, b1_ref, w2_ref, b2_ref, out_ref):
    mask = (tokT_ref[...] != 0).astype(jnp.float32)
    ones = jnp.ones((L, 1), jnp.float32)
    cnt = jax.lax.dot_general(mask, ones, (((0,), (0,)), ((), ())),
                              preferred_element_type=jnp.float32)
    row0 = xtab_ref[0:1, :]
    title = title_ref[:, :EMB]
    sums = sums_ref[:, :EMB]
    text = (sums - (L - cnt) * row0) / jnp.maximum(cnt, 1.0)
    bf = jnp.bfloat16
    h = jnp.dot(title.astype(bf), w1a_ref[...].astype(bf),
                preferred_element_type=jnp.float32)
    h += jnp.dot(text.astype(bf), w1b_ref[...].astype(bf),
                 preferred_element_type=jnp.float32)
    h = jnp.maximum(h + b1_ref[...], 0.0)
    out_ref[...] = (
        jnp.dot(h.astype(bf), w2_ref[...].astype(bf),
                preferred_element_type=jnp.float32) + b2_ref[...]
    )


def _tc_mlp(title_emb, sums, tokT, text_table, w1a, w1b, b1, w2, b2):
    grid = B // BLK
    return pl.pallas_call(
        _mlp_body,
        grid=(grid,),
        in_specs=[
            pl.BlockSpec((BLK, PADW), lambda i: (i, 0)),
            pl.BlockSpec((BLK, PADW), lambda i: (i, 0)),
            pl.BlockSpec((L, BLK), lambda i: (0, i)),
            pl.BlockSpec((8, EMB), lambda i: (0, 0)),
            pl.BlockSpec((EMB, H1), lambda i: (0, 0)),
            pl.BlockSpec((EMB, H1), lambda i: (0, 0)),
            pl.BlockSpec((1, H1), lambda i: (0, 0)),
            pl.BlockSpec((H1, H2), lambda i: (0, 0)),
            pl.BlockSpec((1, H2), lambda i: (0, 0)),
        ],
        out_specs=pl.BlockSpec((BLK, H2), lambda i: (i, 0)),
        out_shape=jax.ShapeDtypeStruct((B, H2), jnp.float32),
    )(title_emb, sums, tokT, text_table, w1a, w1b, b1, w2, b2)


def kernel(title_ids, token_ids, title_table, text_table, W1, b1, W2, b2):
    tid3 = title_ids.astype(jnp.int32).reshape(NW, TID_ROWS_W, 128)
    tokT = token_ids.astype(jnp.int32).T
    sums = _sc_token_pool(tokT, text_table)
    title_emb = _sc_title_gather(tid3, title_table)
    return _tc_mlp(
        title_emb, sums, tokT, text_table,
        W1[:EMB], W1[EMB:], b1.reshape(1, H1), W2, b2.reshape(1, H2),
    )
